# baseline (speedup 1.0000x reference)
import functools

import jax
import jax.numpy as jnp
from jax import lax
from jax.experimental import pallas as pl
from jax.experimental.pallas import tpu as pltpu
from jax.experimental.pallas import tpu_sc as plsc

B, H, W = 32, 1024, 1024
P = 128
L = 16


def kernel(x, probe_x, probe_y):
    x_flat = x.reshape(B * H * W)
    mesh = plsc.VectorSubcoreMesh(core_axis_name="c", subcore_axis_name="s")

    @functools.partial(
        pl.kernel,
        mesh=mesh,
        out_type=jax.ShapeDtypeStruct((B, P), jnp.float32),
        scratch_types=[
            pltpu.VMEM((P,), jnp.int32),
            pltpu.VMEM((P,), jnp.int32),
            pltpu.VMEM((P,), jnp.int32),
            pltpu.VMEM((P,), jnp.float32),
            pltpu.SemaphoreType.DMA,
        ],
    )
    def probe_gather(x_hbm, px_hbm, py_hbm, out_hbm, px_v, py_v, idx_v, val_v, sem):
        wid = lax.axis_index("s") * 2 + lax.axis_index("c")
        pltpu.sync_copy(px_hbm, px_v)
        pltpu.sync_copy(py_hbm, py_v)
        base = wid * (H * W)
        for i in range(P // L):
            sl = pl.ds(i * L, L)
            idx_v[sl] = px_v[sl] * W + py_v[sl] + base
        pltpu.async_copy(x_hbm.at[idx_v], val_v, sem).wait()
        pltpu.sync_copy(val_v, out_hbm.at[wid])

    return probe_gather(x_flat, probe_x, probe_y)


# speedup vs baseline: 4.0778x; 4.0778x over previous
import functools

import jax
import jax.numpy as jnp
from jax import lax
from jax.experimental import pallas as pl
from jax.experimental.pallas import tpu as pltpu
from jax.experimental.pallas import tpu_sc as plsc

B, H, W = 32, 1024, 1024
P = 128
L = 16
CHUNK = 64


def kernel(x, probe_x, probe_y):
    x2 = x.reshape(B * H, W)
    mesh = plsc.VectorSubcoreMesh(core_axis_name="c", subcore_axis_name="s")

    @functools.partial(
        pl.kernel,
        mesh=mesh,
        out_type=jax.ShapeDtypeStruct((B, P), jnp.float32),
        scratch_types=[
            pltpu.VMEM((P,), jnp.int32),
            pltpu.VMEM((P,), jnp.int32),
            pltpu.VMEM((P,), jnp.int32),
            pltpu.VMEM((CHUNK, W), jnp.float32),
            pltpu.VMEM((P,), jnp.float32),
            pltpu.SemaphoreType.DMA,
        ],
        compiler_params=pltpu.CompilerParams(needs_layout_passes=False),
    )
    def probe_gather(x_hbm, px_hbm, py_hbm, out_hbm, px_v, py_v, row_v, buf_v,
                     val_v, sem):
        wid = lax.axis_index("s") * 2 + lax.axis_index("c")
        pltpu.sync_copy(px_hbm, px_v)
        pltpu.sync_copy(py_hbm, py_v)
        base = wid * H
        for i in range(P // L):
            sl = pl.ds(i * L, L)
            row_v[sl] = px_v[sl] + base
        lane_ids = lax.iota(jnp.int32, L)
        for c in range(P // CHUNK):
            pltpu.async_copy(
                x_hbm.at[row_v.at[pl.ds(c * CHUNK, CHUNK)]], buf_v, sem
            ).wait()
            for j in range(CHUNK // L):
                sl = pl.ds(c * CHUNK + j * L, L)
                val_v[sl] = plsc.load_gather(
                    buf_v, [lane_ids + j * L, py_v[sl]]
                )
        pltpu.sync_copy(val_v, out_hbm.at[wid])

    return probe_gather(x2, probe_x, probe_y)


# speedup vs baseline: 5.1568x; 1.2646x over previous
import functools

import jax
import jax.numpy as jnp
from jax import lax
from jax.experimental import pallas as pl
from jax.experimental.pallas import tpu as pltpu
from jax.experimental.pallas import tpu_sc as plsc

B, H, W = 32, 1024, 1024
P = 128
L = 16
LANES = 128
SUBL = 8


def kernel(x, probe_x, probe_y):
    n_tile_rows = B * H // SUBL
    xg = (
        x.reshape(n_tile_rows, SUBL, W // LANES, LANES)
        .transpose(0, 2, 1, 3)
        .reshape(B * H * W // LANES, LANES)
    )
    mesh = plsc.VectorSubcoreMesh(core_axis_name="c", subcore_axis_name="s")

    @functools.partial(
        pl.kernel,
        mesh=mesh,
        out_type=jax.ShapeDtypeStruct((B, P), jnp.float32),
        scratch_types=[
            pltpu.VMEM((P,), jnp.int32),
            pltpu.VMEM((P,), jnp.int32),
            pltpu.VMEM((P,), jnp.int32),
            pltpu.VMEM((P, LANES), jnp.float32),
            pltpu.VMEM((P,), jnp.float32),
            pltpu.SemaphoreType.DMA,
        ],
        compiler_params=pltpu.CompilerParams(needs_layout_passes=False),
    )
    def probe_gather(x_hbm, px_hbm, py_hbm, out_hbm, px_v, py_v, row_v, buf_v,
                     val_v, sem):
        wid = lax.axis_index("s") * 2 + lax.axis_index("c")
        pltpu.sync_copy(px_hbm, px_v)
        pltpu.sync_copy(py_hbm, py_v)
        base = wid * (H * W // LANES)
        for i in range(P // L):
            sl = pl.ds(i * L, L)
            px, py = px_v[sl], py_v[sl]
            row_v[sl] = (
                base
                + (px >> 3) * (SUBL * W // LANES)
                + (py >> 7) * SUBL
                + (px & 7)
            )
        pltpu.async_copy(x_hbm.at[row_v], buf_v, sem).wait()
        lane_ids = lax.iota(jnp.int32, L)
        for j in range(P // L):
            sl = pl.ds(j * L, L)
            val_v[sl] = plsc.load_gather(
                buf_v, [lane_ids + j * L, py_v[sl] & 127]
            )
        pltpu.sync_copy(val_v, out_hbm.at[wid])

    return probe_gather(xg, probe_x, probe_y)
